# Initial kernel scaffold; baseline (speedup 1.0000x reference)
import jax
import jax.numpy as jnp
from jax.experimental import pallas as pl


def kernel(z, emb_weight):
    raise NotImplementedError("write your pallas kernel here")


# speedup vs baseline: 1.5531x; 1.5531x over previous
import functools

import jax
import jax.numpy as jnp
from jax.experimental import pallas as pl
from jax.experimental.pallas import tpu as pltpu

K = 1024
D = 64
N = 16384
BN = 1024
NSTEPS = N // BN


def _vq_kernel(zf_ref, zo_ref, emb_ref, embt_ref,
               enc_ref, zq_ref, idx_ref, loss_ref, perp_ref,
               counts_ref, sse_ref):
    step = pl.program_id(0)

    zf = zf_ref[...]
    emb = emb_ref[...]
    embt = embt_ref[...]

    w = zf * zf
    acc = w[:, 0:8]
    for i in range(1, 8):
        acc = acc + w[:, 8 * i:8 * i + 8]
    acc = acc[:, :4] + acc[:, 4:]
    acc = acc[:, :2] + acc[:, 2:]
    z2 = acc[:, 0:1] + acc[:, 1:2]

    t = embt * embt
    te = t[0:8, :]
    for i in range(1, 8):
        te = te + t[8 * i:8 * i + 8, :]
    te = te[:4, :] + te[4:, :]
    te = te[:2, :] + te[2:, :]
    e2 = te[0:1, :] + te[1:2, :]
    cross = jax.lax.dot_general(
        zf.astype(jnp.bfloat16), emb.astype(jnp.bfloat16),
        (((1,), (1,)), ((), ())),
        preferred_element_type=jnp.float32)
    dist = z2 + e2 - 2.0 * cross

    lanes = jax.lax.broadcasted_iota(jnp.int32, (BN, K), 1)
    dmin = jnp.min(dist, axis=1, keepdims=True)
    idx = jnp.min(jnp.where(dist == dmin, lanes, K), axis=1).astype(jnp.int32)
    idx_ref[...] = idx[:, None]

    onehot = jnp.where(lanes == idx[:, None], 1.0, 0.0).astype(jnp.float32)
    enc_ref[...] = onehot

    zq = jax.lax.dot_general(
        onehot.astype(jnp.bfloat16), emb.astype(jnp.bfloat16),
        (((1,), (0,)), ((), ())),
        preferred_element_type=jnp.float32)
    zq_ref[...] = zq

    @pl.when(step == 0)
    def _init():
        counts_ref[...] = jnp.zeros_like(counts_ref)
        sse_ref[...] = jnp.zeros_like(sse_ref)

    counts_ref[...] += jnp.sum(onehot, axis=0, keepdims=True)
    diff = zq - zo_ref[...]
    sse_ref[...] += jnp.sum(diff * diff)[None, None]

    @pl.when(step == NSTEPS - 1)
    def _finish():
        sse = sse_ref[0, 0]
        loss_ref[...] = ((1.0 + 0.5) * sse / jnp.float32(N * D))[None, None]
        e_mean = counts_ref[...] / jnp.float32(N)
        ent = -jnp.sum(e_mean * jnp.log(e_mean + 1e-10))
        perp_ref[...] = jnp.exp(ent)[None, None]


@functools.partial(jax.jit, static_argnames=())
def kernel(z, emb_weight):
    B, C, H, W = z.shape
    z_flat = jnp.transpose(z, (0, 2, 3, 1)).reshape(N, D)
    z_orig = z.reshape(N, D)

    grid = (NSTEPS,)
    out = pl.pallas_call(
        _vq_kernel,
        grid=grid,
        in_specs=[
            pl.BlockSpec((BN, D), lambda i: (i, 0)),
            pl.BlockSpec((BN, D), lambda i: (i, 0)),
            pl.BlockSpec((K, D), lambda i: (0, 0)),
            pl.BlockSpec((D, K), lambda i: (0, 0)),
        ],
        out_specs=[
            pl.BlockSpec((BN, K), lambda i: (i, 0)),
            pl.BlockSpec((BN, D), lambda i: (i, 0)),
            pl.BlockSpec((BN, 1), lambda i: (i, 0)),
            pl.BlockSpec((1, 1), lambda i: (0, 0)),
            pl.BlockSpec((1, 1), lambda i: (0, 0)),
        ],
        out_shape=[
            jax.ShapeDtypeStruct((N, K), jnp.float32),
            jax.ShapeDtypeStruct((N, D), jnp.float32),
            jax.ShapeDtypeStruct((N, 1), jnp.int32),
            jax.ShapeDtypeStruct((1, 1), jnp.float32),
            jax.ShapeDtypeStruct((1, 1), jnp.float32),
        ],
        scratch_shapes=[
            pltpu.VMEM((1, K), jnp.float32),
            pltpu.VMEM((1, 1), jnp.float32),
        ],
    )(z_flat, z_orig, emb_weight, emb_weight.T)

    min_encodings, zq_flat, encoding_indices, loss, perplexity = out
    z_q = zq_flat.reshape(B, D, H, W)
    return (z_q, perplexity[0, 0], encoding_indices,
            min_encodings, loss[0, 0])
